# baseline (speedup 1.0000x reference)
import jax
import jax.numpy as jnp
from jax.experimental import pallas as pl
from jax.experimental.pallas import tpu as pltpu

N = 2048
E = 4
BI = 256
BJ = 256
CW = 128
JG = 32


def _pass_a(adj_ref, e_ref, x0j_ref, x0i_ref, R_ref, Wb0_ref, be0_ref,
            Wb1_ref, be1_ref, S_ref, Wn0_ref, We0_ref, bn0_ref,
            x1_ref, a1_ref, a2_ref, acc0, acc1, acc2, h0acc):
    j = pl.program_id(1)
    nj = pl.num_programs(1)

    @pl.when(j == 0)
    def _init():
        acc0[...] = jnp.zeros_like(acc0)
        acc1[...] = jnp.zeros_like(acc1)
        acc2[...] = jnp.zeros_like(acc2)
        h0acc[...] = jnp.zeros_like(h0acc)

    adj_t = adj_ref[...]
    R = R_ref[...]
    Wb0 = Wb0_ref[...]
    Wb1 = Wb1_ref[...]
    be0 = be0_ref[...]
    be1 = be1_ref[...]

    a0l = acc0[...]
    a1l = acc1[...]
    a2l = acc2[...]
    for c in range(4 * BJ // CW):
        ec = e_ref[:, CW * c:CW * (c + 1)]
        arep = jnp.dot(adj_t[:, JG * c:JG * (c + 1)], R,
                       preferred_element_type=jnp.float32)
        a0l = a0l + arep * ec
        e1c = jnp.maximum(jnp.dot(ec, Wb0, preferred_element_type=jnp.float32)
                          + be0, 0.0)
        a1l = a1l + arep * e1c
        e2c = jnp.maximum(jnp.dot(e1c, Wb1, preferred_element_type=jnp.float32)
                          + be1, 0.0)
        a2l = a2l + arep * e2c
    acc0[...] = a0l
    acc1[...] = a1l
    acc2[...] = a2l

    h0acc[...] += jnp.dot(adj_t, x0j_ref[...], preferred_element_type=jnp.float32)

    @pl.when(j == nj - 1)
    def _fin():
        S = S_ref[...]
        ea0 = jnp.dot(acc0[...], S, preferred_element_type=jnp.float32)
        h = x0i_ref[...] + h0acc[...]
        x1 = jnp.dot(h, Wn0_ref[...], preferred_element_type=jnp.float32)
        x1 = x1 + jnp.dot(ea0, We0_ref[...], preferred_element_type=jnp.float32)
        x1_ref[...] = jnp.maximum(x1 + bn0_ref[...], 0.0)
        a1_ref[...] = jnp.dot(acc1[...], S, preferred_element_type=jnp.float32)
        a2_ref[...] = jnp.dot(acc2[...], S, preferred_element_type=jnp.float32)


def _pass_b(adj_ref, x1j_ref, x1i_ref, a1_ref, a2_ref, Wn1_ref, We1_ref,
            bn1_ref, Wr_ref, Wre_ref, br_ref, out_ref, h1acc):
    j = pl.program_id(1)
    nj = pl.num_programs(1)

    @pl.when(j == 0)
    def _init():
        h1acc[...] = jnp.zeros_like(h1acc)

    h1acc[...] += jnp.dot(adj_ref[...], x1j_ref[...],
                          preferred_element_type=jnp.float32)

    @pl.when(j == nj - 1)
    def _fin():
        h = x1i_ref[...] + h1acc[...]
        x2 = jnp.dot(h, Wn1_ref[...], preferred_element_type=jnp.float32)
        x2 = x2 + jnp.dot(a1_ref[...], We1_ref[...],
                          preferred_element_type=jnp.float32)
        x2 = jnp.maximum(x2 + bn1_ref[...], 0.0)
        out = jnp.dot(x2, Wr_ref[...], preferred_element_type=jnp.float32)
        out = out + jnp.dot(a2_ref[...], Wre_ref[...],
                            preferred_element_type=jnp.float32)
        out_ref[...] = out + br_ref[...]


def kernel(node_features, edge_features, adj, Wn0, We0, bn0, Wee0, be0,
           Wn1, We1, bn1, Wee1, be1, Wr, Wre, br):
    f32 = jnp.float32
    e2d = edge_features.reshape(N, N * E)

    eyeg = jnp.eye(JG, dtype=f32)
    R = jnp.kron(eyeg, jnp.ones((1, E), dtype=f32))
    Wb0 = jnp.kron(eyeg, Wee0)
    Wb1 = jnp.kron(eyeg, Wee1)
    S = jnp.kron(jnp.ones((JG, 1), dtype=f32), jnp.eye(E, dtype=f32))
    be0r = jnp.tile(be0, JG).reshape(1, CW)
    be1r = jnp.tile(be1, JG).reshape(1, CW)
    bn0r = bn0.reshape(1, -1)
    bn1r = bn1.reshape(1, -1)
    brr = br.reshape(1, -1)

    gi, gj = N // BI, N // BJ
    x1, a1, a2 = pl.pallas_call(
        _pass_a,
        grid=(gi, gj),
        in_specs=[
            pl.BlockSpec((BI, BJ), lambda i, j: (i, j)),
            pl.BlockSpec((BI, E * BJ), lambda i, j: (i, j)),
            pl.BlockSpec((BJ, 128), lambda i, j: (j, 0)),
            pl.BlockSpec((BI, 128), lambda i, j: (i, 0)),
            pl.BlockSpec((JG, CW), lambda i, j: (0, 0)),
            pl.BlockSpec((CW, CW), lambda i, j: (0, 0)),
            pl.BlockSpec((1, CW), lambda i, j: (0, 0)),
            pl.BlockSpec((CW, CW), lambda i, j: (0, 0)),
            pl.BlockSpec((1, CW), lambda i, j: (0, 0)),
            pl.BlockSpec((CW, E), lambda i, j: (0, 0)),
            pl.BlockSpec((128, 256), lambda i, j: (0, 0)),
            pl.BlockSpec((E, 256), lambda i, j: (0, 0)),
            pl.BlockSpec((1, 256), lambda i, j: (0, 0)),
        ],
        out_specs=[
            pl.BlockSpec((BI, 256), lambda i, j: (i, 0)),
            pl.BlockSpec((BI, E), lambda i, j: (i, 0)),
            pl.BlockSpec((BI, E), lambda i, j: (i, 0)),
        ],
        out_shape=[
            jax.ShapeDtypeStruct((N, 256), f32),
            jax.ShapeDtypeStruct((N, E), f32),
            jax.ShapeDtypeStruct((N, E), f32),
        ],
        scratch_shapes=[
            pltpu.VMEM((BI, CW), f32),
            pltpu.VMEM((BI, CW), f32),
            pltpu.VMEM((BI, CW), f32),
            pltpu.VMEM((BI, 128), f32),
        ],
    )(adj, e2d, node_features, node_features, R, Wb0, be0r, Wb1, be1r, S,
      Wn0, We0, bn0r)

    out = pl.pallas_call(
        _pass_b,
        grid=(gi, gj),
        in_specs=[
            pl.BlockSpec((BI, BJ), lambda i, j: (i, j)),
            pl.BlockSpec((BJ, 256), lambda i, j: (j, 0)),
            pl.BlockSpec((BI, 256), lambda i, j: (i, 0)),
            pl.BlockSpec((BI, E), lambda i, j: (i, 0)),
            pl.BlockSpec((BI, E), lambda i, j: (i, 0)),
            pl.BlockSpec((256, 256), lambda i, j: (0, 0)),
            pl.BlockSpec((E, 256), lambda i, j: (0, 0)),
            pl.BlockSpec((1, 256), lambda i, j: (0, 0)),
            pl.BlockSpec((256, 64), lambda i, j: (0, 0)),
            pl.BlockSpec((E, 64), lambda i, j: (0, 0)),
            pl.BlockSpec((1, 64), lambda i, j: (0, 0)),
        ],
        out_specs=pl.BlockSpec((BI, 64), lambda i, j: (i, 0)),
        out_shape=jax.ShapeDtypeStruct((N, 64), f32),
        scratch_shapes=[
            pltpu.VMEM((BI, 256), f32),
        ],
    )(adj, x1, x1, a1, a2, Wn1, We1, bn1r, Wr, Wre, brr)
    return out
